# baseline (speedup 1.0000x reference)
import functools

import jax
import jax.numpy as jnp
from jax import lax
from jax.experimental import pallas as pl
from jax.experimental.pallas import tpu as pltpu
from jax.experimental.pallas import tpu_sc as plsc

N = 10000
E = 320000
NT = 32
EPT = E // NT
NV = EPT // 16

_mesh = plsc.VectorSubcoreMesh(core_axis_name="c", subcore_axis_name="s")
_sc_params = pltpu.CompilerParams(needs_layout_passes=False)


EBUF = EPT + 112


def _edge_window(wid):
    off = 16 * lax.rem(wid, 8)
    start = pl.multiple_of(wid * EPT - off, 128)
    return start, off


def _deg_body(ei, out, ei_v, acc_v):
    cid = lax.axis_index("c")
    sid = lax.axis_index("s")
    wid = cid * 16 + sid
    start, off = _edge_window(wid)
    pltpu.sync_copy(ei.at[:, pl.ds(start, EBUF)], ei_v)

    @plsc.parallel_loop(0, NV, unroll=16)
    def zero(i):
        acc_v[pl.ds(i * 16, 16)] = jnp.zeros((16,), jnp.float32)

    ones = jnp.full((16,), 1.0, jnp.float32)

    @plsc.parallel_loop(0, NV, unroll=16)
    def step(v):
        d16 = ei_v[1, pl.ds(off + v * 16, 16)]
        plsc.addupdate_scatter(acc_v, [d16], ones)
    pltpu.sync_copy(acc_v, out.at[wid])


_deg_kernel = functools.partial(
    pl.kernel,
    out_type=jax.ShapeDtypeStruct((NT, N), jnp.float32),
    mesh=_mesh,
    compiler_params=_sc_params,
    scratch_types=[
        pltpu.VMEM((2, EBUF), jnp.int32),
        pltpu.VMEM((N,), jnp.float32),
    ],
)(_deg_body)


def _prop_body(C, tbl_h, ei, out, tbl_v, ei_v, acc_v):
    cid = lax.axis_index("c")
    sid = lax.axis_index("s")
    wid = cid * 16 + sid
    start, off = _edge_window(wid)
    pltpu.sync_copy(ei.at[:, pl.ds(start, EBUF)], ei_v)
    pltpu.sync_copy(tbl_h, tbl_v)

    @plsc.parallel_loop(0, NV, unroll=16)
    def zero(i):
        for c in range(C):
            acc_v[c, pl.ds(i * 16, 16)] = jnp.zeros((16,), jnp.float32)

    @plsc.parallel_loop(0, NV, unroll=16)
    def step(v):
        s16 = ei_v[0, pl.ds(off + v * 16, 16)]
        d16 = ei_v[1, pl.ds(off + v * 16, 16)]
        for c in range(C):
            cc = jnp.full((16,), c, jnp.int32)
            g = plsc.load_gather(tbl_v, [cc, s16])
            plsc.addupdate_scatter(acc_v, [cc, d16], g)
    pltpu.sync_copy(acc_v, out.at[wid])


def _make_prop(C):
    return functools.partial(
        pl.kernel,
        out_type=jax.ShapeDtypeStruct((NT, C, N), jnp.float32),
        mesh=_mesh,
        compiler_params=_sc_params,
        scratch_types=[
            pltpu.VMEM((C, N), jnp.float32),
            pltpu.VMEM((2, EBUF), jnp.int32),
            pltpu.VMEM((C, N), jnp.float32),
        ],
    )(functools.partial(_prop_body, C))


_prop4 = _make_prop(4)
_prop2 = _make_prop(2)


def _t1a_body(x, w1t, out_h1):
    out_h1[...] = lax.dot_general(
        w1t[...], x[...], (((1,), (1,)), ((), ())),
        preferred_element_type=jnp.float32)


def _t1b_body(degp, h1, out_dinv, out_h1s):
    deg = jnp.sum(degp[...], axis=0, keepdims=True) + 1.0
    dinv = lax.rsqrt(jnp.maximum(deg, 1.0))
    out_dinv[...] = dinv
    out_h1s[...] = h1[...] * dinv


def _t2_body(p1, h1s, dinv, w2t, b1c, out):
    z1 = jnp.maximum(
        (jnp.sum(p1[...], axis=0) + h1s[...]) * dinv[...] + b1c[...], 0.0)
    h2 = jnp.dot(w2t[...], z1, preferred_element_type=jnp.float32)
    out[...] = h2 * dinv[...]


def _t3_body(p2, h2s, dinv, b2c, out):
    z2 = (jnp.sum(p2[...], axis=0) + h2s[...]) * dinv[...] + b2c[...]
    out[...] = z2 * dinv[...]


def _t4_body(p3, q, dinv, w3t, b3c, w4t, out):
    az = (jnp.sum(p3[...], axis=0) + q[...]) * dinv[...]
    h3 = jnp.maximum(
        jnp.dot(w3t[...], az, preferred_element_type=jnp.float32) + b3c[...], 0.0)
    h4 = jnp.dot(w4t[...], h3, preferred_element_type=jnp.float32)
    out[...] = h4 * dinv[...]


def _t5_body(p4, h4s, dinv, b4c, out):
    out[...] = (jnp.sum(p4[...], axis=0) + h4s[...]) * dinv[...] + b4c[...]


def _tc(body, out_ch):
    if isinstance(out_ch, tuple):
        outs = tuple(jax.ShapeDtypeStruct((c, N), jnp.float32) for c in out_ch)
    else:
        outs = jax.ShapeDtypeStruct((out_ch, N), jnp.float32)
    return pl.pallas_call(body, out_shape=outs)


def kernel(x, edge_index, W1, b1, W2, b2, W3, b3, W4, b4):
    ei = edge_index.astype(jnp.int32)

    degp = _deg_kernel(ei)
    h1 = _tc(_t1a_body, 4)(x, W1.T)
    dinv, h1s = _tc(_t1b_body, (1, 4))(degp, h1)
    p1 = _prop4(h1s, ei)
    h2s = _tc(_t2_body, 2)(p1, h1s, dinv, W2.T, b1.reshape(4, 1))
    p2 = _prop2(h2s, ei)
    q = _tc(_t3_body, 2)(p2, h2s, dinv, b2.reshape(2, 1))
    p3 = _prop2(q, ei)
    h4s = _tc(_t4_body, 2)(p3, q, dinv, W3.T, b3.reshape(64, 1), W4.T)
    p4 = _prop2(h4s, ei)
    c = _tc(_t5_body, 2)(p4, h4s, dinv, b4.reshape(2, 1))
    return c.T


# speedup vs baseline: 1.0692x; 1.0692x over previous
import functools

import jax
import jax.numpy as jnp
from jax import lax
from jax.experimental import pallas as pl
from jax.experimental.pallas import tpu as pltpu
from jax.experimental.pallas import tpu_sc as plsc

N = 10000
E = 320000
NT = 32
EPT = E // NT
NV = EPT // 16

_mesh = plsc.VectorSubcoreMesh(core_axis_name="c", subcore_axis_name="s")
_sc_params = pltpu.CompilerParams(needs_layout_passes=False)


EBUF = EPT + 112


def _edge_window(wid):
    off = 16 * lax.rem(wid, 8)
    start = pl.multiple_of(wid * EPT - off, 128)
    return start, off


def _deg_body(ei, out, ei_v, acc_v, sem):
    cid = lax.axis_index("c")
    sid = lax.axis_index("s")
    wid = cid * 16 + sid
    start, off = _edge_window(wid)
    cp = pltpu.async_copy(ei.at[:, pl.ds(start, EBUF)], ei_v, sem)

    @plsc.parallel_loop(0, NV, unroll=8)
    def zero(i):
        acc_v[pl.ds(i * 16, 16)] = jnp.zeros((16,), jnp.float32)

    cp.wait()
    ones = jnp.full((16,), 1.0, jnp.float32)

    @plsc.parallel_loop(0, NV, unroll=8)
    def step(v):
        d16 = ei_v[1, pl.ds(off + v * 16, 16)]
        plsc.addupdate_scatter(acc_v, [d16], ones)
    pltpu.sync_copy(acc_v, out.at[wid])


_deg_kernel = functools.partial(
    pl.kernel,
    out_type=jax.ShapeDtypeStruct((NT, N), jnp.float32),
    mesh=_mesh,
    compiler_params=_sc_params,
    scratch_types=[
        pltpu.VMEM((2, EBUF), jnp.int32),
        pltpu.VMEM((N,), jnp.float32),
        pltpu.SemaphoreType.DMA,
    ],
)(_deg_body)


def _prop_body(C, tbl_h, ei, out, tbl_v, ei_v, acc_v, sem, sem2):
    cid = lax.axis_index("c")
    sid = lax.axis_index("s")
    wid = cid * 16 + sid
    start, off = _edge_window(wid)
    cp1 = pltpu.async_copy(ei.at[:, pl.ds(start, EBUF)], ei_v, sem)
    cp2 = pltpu.async_copy(tbl_h, tbl_v, sem2)

    @plsc.parallel_loop(0, NV, unroll=8)
    def zero(i):
        for c in range(C):
            acc_v[c, pl.ds(i * 16, 16)] = jnp.zeros((16,), jnp.float32)

    cp1.wait()
    cp2.wait()

    @plsc.parallel_loop(0, NV, unroll=8)
    def step(v):
        s16 = ei_v[0, pl.ds(off + v * 16, 16)]
        d16 = ei_v[1, pl.ds(off + v * 16, 16)]
        for c in range(C):
            cc = jnp.full((16,), c, jnp.int32)
            g = plsc.load_gather(tbl_v, [cc, s16])
            plsc.addupdate_scatter(acc_v, [cc, d16], g)
    pltpu.sync_copy(acc_v, out.at[wid])


def _make_prop(C):
    return functools.partial(
        pl.kernel,
        out_type=jax.ShapeDtypeStruct((NT, C, N), jnp.float32),
        mesh=_mesh,
        compiler_params=_sc_params,
        scratch_types=[
            pltpu.VMEM((C, N), jnp.float32),
            pltpu.VMEM((2, EBUF), jnp.int32),
            pltpu.VMEM((C, N), jnp.float32),
            pltpu.SemaphoreType.DMA,
            pltpu.SemaphoreType.DMA,
        ],
    )(functools.partial(_prop_body, C))


_prop4 = _make_prop(4)
_prop2 = _make_prop(2)


def _t1a_body(x, w1t, out_h1):
    out_h1[...] = lax.dot_general(
        w1t[...], x[...], (((1,), (1,)), ((), ())),
        preferred_element_type=jnp.float32)


def _t1b_body(degp, h1, out_dinv, out_h1s):
    deg = jnp.sum(degp[...], axis=0, keepdims=True) + 1.0
    dinv = lax.rsqrt(jnp.maximum(deg, 1.0))
    out_dinv[...] = dinv
    out_h1s[...] = h1[...] * dinv


def _t2_body(p1, h1s, dinv, w2t, b1c, out):
    z1 = jnp.maximum(
        (jnp.sum(p1[...], axis=0) + h1s[...]) * dinv[...] + b1c[...], 0.0)
    h2 = jnp.dot(w2t[...], z1, preferred_element_type=jnp.float32)
    out[...] = h2 * dinv[...]


def _t3_body(p2, h2s, dinv, b2c, out):
    z2 = (jnp.sum(p2[...], axis=0) + h2s[...]) * dinv[...] + b2c[...]
    out[...] = z2 * dinv[...]


def _t4_body(p3, q, dinv, w3t, b3c, w4t, out):
    az = (jnp.sum(p3[...], axis=0) + q[...]) * dinv[...]
    h3 = jnp.maximum(
        jnp.dot(w3t[...], az, preferred_element_type=jnp.float32) + b3c[...], 0.0)
    h4 = jnp.dot(w4t[...], h3, preferred_element_type=jnp.float32)
    out[...] = h4 * dinv[...]


def _t5_body(p4, h4s, dinv, b4c, out):
    out[...] = (jnp.sum(p4[...], axis=0) + h4s[...]) * dinv[...] + b4c[...]


def _tc(body, out_ch):
    if isinstance(out_ch, tuple):
        outs = tuple(jax.ShapeDtypeStruct((c, N), jnp.float32) for c in out_ch)
    else:
        outs = jax.ShapeDtypeStruct((out_ch, N), jnp.float32)
    return pl.pallas_call(body, out_shape=outs)


def kernel(x, edge_index, W1, b1, W2, b2, W3, b3, W4, b4):
    ei = edge_index.astype(jnp.int32)

    degp = _deg_kernel(ei)
    h1 = _tc(_t1a_body, 4)(x, W1.T)
    dinv, h1s = _tc(_t1b_body, (1, 4))(degp, h1)
    p1 = _prop4(h1s, ei)
    h2s = _tc(_t2_body, 2)(p1, h1s, dinv, W2.T, b1.reshape(4, 1))
    p2 = _prop2(h2s, ei)
    q = _tc(_t3_body, 2)(p2, h2s, dinv, b2.reshape(2, 1))
    p3 = _prop2(q, ei)
    h4s = _tc(_t4_body, 2)(p3, q, dinv, W3.T, b3.reshape(64, 1), W4.T)
    p4 = _prop2(h4s, ei)
    c = _tc(_t5_body, 2)(p4, h4s, dinv, b4.reshape(2, 1))
    return c.T
